# baseline (device time: 13022 ns/iter reference)
import jax
import jax.numpy as jnp
from jax import lax
from jax.experimental import pallas as pl
from jax.experimental.pallas import tpu as pltpu

C = 2


def kernel(x):
    m, n = x.shape
    h = m // 2
    q = h // C

    def body(x_ref, out_ref, s1, r1, s2, r2, send_sems, recv_sems, copy_sems):
        my = lax.axis_index("i")
        p1 = my ^ 1
        p2 = 3 - my

        barrier_sem = pltpu.get_barrier_semaphore()
        for nbr in (p1, p2):
            pl.semaphore_signal(
                barrier_sem, inc=1,
                device_id=(nbr,), device_id_type=pl.DeviceIdType.MESH,
            )

        def partners(s):
            return (p1, p2) if s == 0 else (p2, p1)

        def rows(s, c):
            return pl.ds(s * h + c * q, q)

        for c in range(C):
            for s in range(2):
                i = s * C + c
                s1[i, :, :] = x_ref[rows(s, c), :].astype(jnp.bfloat16)

        pl.semaphore_wait(barrier_sem, 2)

        ph1 = [[None] * C for _ in range(2)]
        ph2 = [[None] * C for _ in range(2)]

        for c in range(C):
            for s in range(2):
                i = s * C + c
                ph1[s][c] = pltpu.make_async_remote_copy(
                    src_ref=s1.at[i], dst_ref=r1.at[i],
                    send_sem=send_sems.at[i], recv_sem=recv_sems.at[i],
                    device_id=(partners(s)[0],),
                    device_id_type=pl.DeviceIdType.MESH,
                )
                ph1[s][c].start()

        for c in range(C):
            for s in range(2):
                i = s * C + c
                ph1[s][c].wait_recv()
                s2[i, :, :] = s1[i, :, :] + r1[i, :, :]
                ph2[s][c] = pltpu.make_async_remote_copy(
                    src_ref=s2.at[i], dst_ref=r2.at[i],
                    send_sem=send_sems.at[2 * C + i],
                    recv_sem=recv_sems.at[2 * C + i],
                    device_id=(partners(s)[1],),
                    device_id_type=pl.DeviceIdType.MESH,
                )
                ph2[s][c].start()

        copies = []
        for c in range(C):
            for s in range(2):
                i = s * C + c
                ph2[s][c].wait_recv()
                ph1[s][c].wait_send()
                s1[i, :, :] = s2[i, :, :] + r2[i, :, :]
                cp = pltpu.make_async_copy(
                    s1.at[i], out_ref.at[rows(s, c), :], copy_sems.at[i]
                )
                cp.start()
                copies.append(cp)
        for cp in copies:
            cp.wait()

        for c in range(C):
            for s in range(2):
                ph2[s][c].wait_send()

    chunks = pltpu.VMEM((2 * C, q, n), jnp.bfloat16)
    return pl.pallas_call(
        body,
        out_shape=jax.ShapeDtypeStruct((m, n), jnp.bfloat16),
        in_specs=[pl.BlockSpec(memory_space=pltpu.VMEM)],
        out_specs=pl.BlockSpec(memory_space=pl.ANY),
        scratch_shapes=[
            chunks, chunks,
            chunks, chunks,
            pltpu.SemaphoreType.DMA((4 * C,)),
            pltpu.SemaphoreType.DMA((4 * C,)),
            pltpu.SemaphoreType.DMA((2 * C,)),
        ],
        compiler_params=pltpu.CompilerParams(collective_id=0),
    )(x)


# device time: 12741 ns/iter; 1.0221x vs baseline; 1.0221x over previous
import jax
import jax.numpy as jnp
from jax import lax
from jax.experimental import pallas as pl
from jax.experimental.pallas import tpu as pltpu

C = 2


def kernel(x):
    m, n = x.shape
    h = m // 2
    q = h // C

    def body(x_ref, out_ref, xv, s1, r1, s2, r2, send_sems, recv_sems, copy_sems):
        my = lax.axis_index("i")
        p1 = my ^ 1
        p2 = 3 - my

        in_dma = pltpu.make_async_copy(x_ref, xv, copy_sems.at[0])
        in_dma.start()

        barrier_sem = pltpu.get_barrier_semaphore()
        for nbr in (p1, p2):
            pl.semaphore_signal(
                barrier_sem, inc=1,
                device_id=(nbr,), device_id_type=pl.DeviceIdType.MESH,
            )
        in_dma.wait()

        def partners(s):
            return (p1, p2) if s == 0 else (p2, p1)

        def rows(s, c):
            return pl.ds(s * h + c * q, q)

        for c in range(C):
            for s in range(2):
                i = s * C + c
                s1[i, :, :] = xv[rows(s, c), :].astype(jnp.bfloat16)

        pl.semaphore_wait(barrier_sem, 2)

        ph1 = [[None] * C for _ in range(2)]
        ph2 = [[None] * C for _ in range(2)]

        for c in range(C):
            for s in range(2):
                i = s * C + c
                ph1[s][c] = pltpu.make_async_remote_copy(
                    src_ref=s1.at[i], dst_ref=r1.at[i],
                    send_sem=send_sems.at[i], recv_sem=recv_sems.at[i],
                    device_id=(partners(s)[0],),
                    device_id_type=pl.DeviceIdType.MESH,
                )
                ph1[s][c].start()

        for c in range(C):
            for s in range(2):
                i = s * C + c
                ph1[s][c].wait_recv()
                s2[i, :, :] = s1[i, :, :] + r1[i, :, :]
                ph2[s][c] = pltpu.make_async_remote_copy(
                    src_ref=s2.at[i], dst_ref=r2.at[i],
                    send_sem=send_sems.at[2 * C + i],
                    recv_sem=recv_sems.at[2 * C + i],
                    device_id=(partners(s)[1],),
                    device_id_type=pl.DeviceIdType.MESH,
                )
                ph2[s][c].start()

        for c in range(C):
            for s in range(2):
                i = s * C + c
                ph2[s][c].wait_recv()
                out_ref[rows(s, c), :] = s2[i, :, :] + r2[i, :, :]

        for c in range(C):
            for s in range(2):
                ph1[s][c].wait_send()
                ph2[s][c].wait_send()

    chunks = pltpu.VMEM((2 * C, q, n), jnp.bfloat16)
    return pl.pallas_call(
        body,
        out_shape=jax.ShapeDtypeStruct((m, n), jnp.bfloat16),
        in_specs=[pl.BlockSpec(memory_space=pl.ANY)],
        out_specs=pl.BlockSpec(memory_space=pltpu.VMEM),
        scratch_shapes=[
            pltpu.VMEM((m, n), jnp.float32),
            chunks, chunks,
            chunks, chunks,
            pltpu.SemaphoreType.DMA((4 * C,)),
            pltpu.SemaphoreType.DMA((4 * C,)),
            pltpu.SemaphoreType.DMA((1,)),
        ],
        compiler_params=pltpu.CompilerParams(collective_id=0),
    )(x)


# device time: 12655 ns/iter; 1.0290x vs baseline; 1.0068x over previous
import jax
import jax.numpy as jnp
from jax import lax
from jax.experimental import pallas as pl
from jax.experimental.pallas import tpu as pltpu

C = 2


def kernel(x):
    m, n = x.shape
    h = m // 2
    q = h // C

    def body(x_ref, out_ref, s1, r1, s2, r2, send_sems, recv_sems):
        my = lax.axis_index("i")
        p1 = my ^ 1
        p2 = 3 - my

        barrier_sem = pltpu.get_barrier_semaphore()
        for nbr in (p1, p2):
            pl.semaphore_signal(
                barrier_sem, inc=1,
                device_id=(nbr,), device_id_type=pl.DeviceIdType.MESH,
            )

        def partners(s):
            return (p1, p2) if s == 0 else (p2, p1)

        def rows(s, c):
            return pl.ds(s * h + c * q, q)

        for c in range(C):
            for s in range(2):
                i = s * C + c
                s1[i, :, :] = x_ref[rows(s, c), :].astype(jnp.bfloat16)

        pl.semaphore_wait(barrier_sem, 2)

        ph1 = [[None] * C for _ in range(2)]
        ph2 = [[None] * C for _ in range(2)]

        for c in range(C):
            for s in range(2):
                i = s * C + c
                ph1[s][c] = pltpu.make_async_remote_copy(
                    src_ref=s1.at[i], dst_ref=r1.at[i],
                    send_sem=send_sems.at[i], recv_sem=recv_sems.at[i],
                    device_id=(partners(s)[0],),
                    device_id_type=pl.DeviceIdType.MESH,
                )
                ph1[s][c].start()

        for c in range(C):
            for s in range(2):
                i = s * C + c
                ph1[s][c].wait_recv()
                s2[i, :, :] = s1[i, :, :] + r1[i, :, :]
                ph2[s][c] = pltpu.make_async_remote_copy(
                    src_ref=s2.at[i], dst_ref=r2.at[i],
                    send_sem=send_sems.at[2 * C + i],
                    recv_sem=recv_sems.at[2 * C + i],
                    device_id=(partners(s)[1],),
                    device_id_type=pl.DeviceIdType.MESH,
                )
                ph2[s][c].start()

        for c in range(C):
            for s in range(2):
                i = s * C + c
                ph2[s][c].wait_recv()
                out_ref[rows(s, c), :] = s2[i, :, :] + r2[i, :, :]

        for c in range(C):
            for s in range(2):
                ph1[s][c].wait_send()
                ph2[s][c].wait_send()

    chunks = pltpu.VMEM((2 * C, q, n), jnp.bfloat16)
    return pl.pallas_call(
        body,
        out_shape=jax.ShapeDtypeStruct((m, n), jnp.bfloat16),
        in_specs=[pl.BlockSpec(memory_space=pltpu.VMEM)],
        out_specs=pl.BlockSpec(memory_space=pltpu.VMEM),
        scratch_shapes=[
            chunks, chunks,
            chunks, chunks,
            pltpu.SemaphoreType.DMA((4 * C,)),
            pltpu.SemaphoreType.DMA((4 * C,)),
        ],
        compiler_params=pltpu.CompilerParams(collective_id=0),
    )(x)
